# Initial kernel scaffold; baseline (speedup 1.0000x reference)
import jax
import jax.numpy as jnp
from jax.experimental import pallas as pl


def kernel(beta, tnet_weights, flat_tnet2pin, pin2node_map):
    raise NotImplementedError("write your pallas kernel here")


# speedup vs baseline: 159.2814x; 159.2814x over previous
import functools

import jax
import jax.numpy as jnp
from jax import lax
from jax.experimental import pallas as pl
from jax.experimental.pallas import tpu as pltpu
from jax.experimental.pallas import tpu_sc as plsc

NT = 2_000_000
NP = 2_000_000
NN = 500_000

N_PAD = 524_288
NC = 2
NS = 16
NW = NC * NS

UNITS = NT // 64
U_HI = 977
U_LO = 976
N_HI = UNITS - U_LO * NW
CHUNK_U = 16
FULL_CHUNKS = U_LO // CHUNK_U

PER_SUB = N_PAD // NS


def _sc_body(w_hbm, pins_hbm, p2n_hbm, out_hbm, acc, pins_v, w_v, nodes_v,
             wexp_v, zbuf, sem):
  c = lax.axis_index("c")
  s = lax.axis_index("s")
  wid = s * NC + c

  iota = lax.iota(jnp.int32, 16)
  half = iota >> 1
  zero16 = jnp.zeros((16,), jnp.float32)

  for g in range(2048 // 16):
    zbuf[pl.ds(g * 16, 16)] = zero16
  zb = s * PER_SUB
  for k in range(PER_SUB // 2048):
    pltpu.sync_copy(zbuf, acc.at[pl.ds(zb + k * 2048, 2048)])
  plsc.subcore_barrier()

  u0 = jnp.where(wid < N_HI, wid * U_HI, N_HI * U_HI + (wid - N_HI) * U_LO)

  def do_units(ubase, nu):
    t0 = pl.multiple_of(ubase * 64, 64)
    npins = nu * 128
    ntn = nu * 64
    pltpu.sync_copy(pins_hbm.at[pl.ds(t0 * 2, npins)],
                    pins_v.at[pl.ds(0, npins)])
    pltpu.sync_copy(w_hbm.at[pl.ds(t0, ntn)], w_v.at[pl.ds(0, ntn)])
    cps = []
    for j in range(nu):
      cp = pltpu.async_copy(
          p2n_hbm.at[pins_v.at[pl.ds(j * 128, 128)]], nodes_v.at[j], sem)
      cps.append(cp)
    for j in range(nu):
      for g in range(8):
        widx = j * 64 + g * 8 + half
        wexp_v[j, pl.ds(g * 16, 16)] = plsc.load_gather(w_v, [widx])
    for cp in cps:
      cp.wait()
    for j in range(nu):
      pltpu.sync_copy(wexp_v.at[j], acc.at[nodes_v.at[j]], add=True)

  def chunk_body(i, carry):
    do_units(u0 + i * CHUNK_U, CHUNK_U)
    return carry

  lax.fori_loop(0, FULL_CHUNKS, chunk_body, 0)

  @pl.when(wid < N_HI)
  def _tail():
    do_units(u0 + FULL_CHUNKS * CHUNK_U, 1)

  plsc.subcore_barrier()
  pltpu.sync_copy(acc.at[pl.ds(zb, PER_SUB)],
                  out_hbm.at[c].at[pl.ds(zb, PER_SUB)])


@jax.jit
def _sc_scatter(tnet_weights, flat_tnet2pin, pin2node_map):
  mesh = plsc.VectorSubcoreMesh(core_axis_name="c", subcore_axis_name="s")
  f = pl.kernel(
      _sc_body,
      out_type=jax.ShapeDtypeStruct((NC, N_PAD), jnp.float32),
      mesh=mesh,
      compiler_params=pltpu.CompilerParams(needs_layout_passes=False),
      scratch_types=[
          pltpu.VMEM_SHARED((N_PAD,), jnp.float32),
          pltpu.VMEM((CHUNK_U * 128,), jnp.int32),
          pltpu.VMEM((CHUNK_U * 64,), jnp.float32),
          pltpu.VMEM((CHUNK_U, 128), jnp.int32),
          pltpu.VMEM((CHUNK_U, 128), jnp.float32),
          pltpu.VMEM((2048,), jnp.float32),
          pltpu.SemaphoreType.DMA,
      ],
  )
  return f(tnet_weights, flat_tnet2pin, pin2node_map)


def _combine_body(beta_ref, p_ref, o_ref):
  o_ref[...] = (p_ref[0] + p_ref[1]) * beta_ref[0]


@jax.jit
def _combine(partials, beta):
  f = pl.pallas_call(
      _combine_body,
      out_shape=jax.ShapeDtypeStruct((N_PAD // 128, 128), jnp.float32),
      in_specs=[
          pl.BlockSpec(memory_space=pltpu.SMEM),
          pl.BlockSpec(memory_space=pltpu.VMEM),
      ],
      out_specs=pl.BlockSpec(memory_space=pltpu.VMEM),
  )
  return f(beta, partials.reshape(NC, N_PAD // 128, 128))


def kernel(beta, tnet_weights, flat_tnet2pin, pin2node_map):
  partials = _sc_scatter(tnet_weights, flat_tnet2pin, pin2node_map)
  out = _combine(partials, beta)
  return out.reshape(-1)[:NN]
